# baseline (device time: 26912 ns/iter reference)
import jax
import jax.numpy as jnp
from jax import lax
from jax.experimental import pallas as pl
from jax.experimental.pallas import tpu as pltpu

B, S, N = 4, 512, 1024
H, D = 8, 64
K = H * D
S_HALF = S // 2
ROWS = 128
SUB = 64
NSUB = ROWS // SUB


def kernel(O, Wo):
    O_t = jnp.transpose(O, (0, 2, 3, 1)).reshape(B, K, S)

    def body(o_hbm, wo_hbm, out_ref, wo_ref, ochunk, ysend, yrecv, zrecv,
             vstage, wo_sem, o_sems, ysend_sems, yrecv_sems, zsend_sems,
             zrecv_sems, out_sems):
        my_x = lax.axis_index("x")
        my_y = lax.axis_index("y")
        my_z = lax.axis_index("z")
        other_y = 1 - my_y
        ypartner = (my_x, other_y, my_z)
        zneighbor = (my_x, my_y, 1 - my_z)

        my_start = my_y * S_HALF
        other_start = other_y * S_HALF
        zc = my_z * ROWS
        oc = (1 - my_z) * ROWS

        wo_copy = pltpu.make_async_copy(wo_hbm, wo_ref, wo_sem)
        wo_copy.start()
        starts = (other_start + zc, my_start + zc, my_start + oc)
        o_copies = {}
        for r, st in enumerate(starts):
            for b in range(B):
                c = pltpu.make_async_copy(
                    o_hbm.at[b, :, pl.ds(st, ROWS)],
                    ochunk.at[r, b],
                    o_sems.at[r, b],
                )
                c.start()
                o_copies[r, b] = c

        barrier = pltpu.get_barrier_semaphore()
        for nbr in (ypartner, zneighbor):
            pl.semaphore_signal(
                barrier, inc=1, device_id=nbr,
                device_id_type=pl.DeviceIdType.MESH,
            )
        pl.semaphore_wait(barrier, 2)

        wo_copy.wait()

        def _chunk(r, b):
            o_copies[r, b].wait()
            return lax.dot_general(
                ochunk[r, b], wo_ref[:, :],
                (((0,), (0,)), ((), ())),
                preferred_element_type=jnp.float32,
            )

        y_rdmas = {}
        for b in range(B):
            part = _chunk(0, b)
            ysend[b] = part.astype(jnp.bfloat16)
            for j in range(NSUB):
                r = pltpu.make_async_remote_copy(
                    src_ref=ysend.at[b, pl.ds(j * SUB, SUB), :],
                    dst_ref=yrecv.at[b, pl.ds(j * SUB, SUB), :],
                    send_sem=ysend_sems.at[b, j],
                    recv_sem=yrecv_sems.at[b, j],
                    device_id=ypartner,
                    device_id_type=pl.DeviceIdType.MESH,
                )
                r.start()
                y_rdmas[b, j] = r

        z_rdmas = {}
        out_copies = []
        for b in range(B):
            own = _chunk(1, b)
            for j in range(NSUB):
                y_rdmas[b, j].wait_recv()
                zr = pltpu.make_async_remote_copy(
                    src_ref=yrecv.at[b, pl.ds(j * SUB, SUB), :],
                    dst_ref=zrecv.at[b, pl.ds(j * SUB, SUB), :],
                    send_sem=zsend_sems.at[b, j],
                    recv_sem=zrecv_sems.at[b, j],
                    device_id=zneighbor,
                    device_id_type=pl.DeviceIdType.MESH,
                )
                zr.start()
                z_rdmas[b, j] = zr
            vstage[0, b] = own + yrecv[b].astype(jnp.float32)
            oc_copy = pltpu.make_async_copy(
                vstage.at[0, b],
                out_ref.at[b, pl.ds(zc, ROWS), :],
                out_sems.at[0, b],
            )
            oc_copy.start()
            out_copies.append(oc_copy)

        for b in range(B):
            own = _chunk(2, b)
            for j in range(NSUB):
                z_rdmas[b, j].wait_recv()
            vstage[1, b] = own + zrecv[b].astype(jnp.float32)
            oc_copy = pltpu.make_async_copy(
                vstage.at[1, b],
                out_ref.at[b, pl.ds(oc, ROWS), :],
                out_sems.at[1, b],
            )
            oc_copy.start()
            out_copies.append(oc_copy)

        for c in out_copies:
            c.wait()
        for b in range(B):
            for j in range(NSUB):
                y_rdmas[b, j].wait_send()
                z_rdmas[b, j].wait_send()

    return pl.pallas_call(
        body,
        out_shape=jax.ShapeDtypeStruct((B, S_HALF, N), jnp.float32),
        in_specs=[
            pl.BlockSpec(memory_space=pltpu.MemorySpace.HBM),
            pl.BlockSpec(memory_space=pltpu.MemorySpace.HBM),
        ],
        out_specs=pl.BlockSpec(memory_space=pltpu.MemorySpace.HBM),
        scratch_shapes=[
            pltpu.VMEM((K, N), jnp.float32),
            pltpu.VMEM((3, B, K, ROWS), jnp.float32),
            pltpu.VMEM((B, ROWS, N), jnp.bfloat16),
            pltpu.VMEM((B, ROWS, N), jnp.bfloat16),
            pltpu.VMEM((B, ROWS, N), jnp.bfloat16),
            pltpu.VMEM((2, B, ROWS, N), jnp.float32),
            pltpu.SemaphoreType.DMA,
            pltpu.SemaphoreType.DMA((3, B)),
            pltpu.SemaphoreType.DMA((B, NSUB)),
            pltpu.SemaphoreType.DMA((B, NSUB)),
            pltpu.SemaphoreType.DMA((B, NSUB)),
            pltpu.SemaphoreType.DMA((B, NSUB)),
            pltpu.SemaphoreType.DMA((2, B)),
        ],
        compiler_params=pltpu.CompilerParams(collective_id=0),
    )(O_t, Wo)


# device time: 23924 ns/iter; 1.1249x vs baseline; 1.1249x over previous
import jax
import jax.numpy as jnp
from jax import lax
from jax.experimental import pallas as pl
from jax.experimental.pallas import tpu as pltpu

B, S, N = 4, 512, 1024
H, D = 8, 64
K = H * D
S_HALF = S // 2
QR = 64


def kernel(O, Wo):
    O_t = jnp.transpose(O, (0, 2, 3, 1)).reshape(B, K, S)

    def body(o_ref, wo_ref, out_ref, ysend, yrecv, xrecv, zrecv, drecv,
             ys_sems, yr_sems, xs_sems, xr_sems, zs_sems, zr_sems,
             ds_sems, dr_sems):
        my_x = lax.axis_index("x")
        my_y = lax.axis_index("y")
        my_z = lax.axis_index("z")
        ypartner = (my_x, 1 - my_y, my_z)
        xneighbor = (1 - my_x, my_y, my_z)
        zneighbor = (my_x, my_y, 1 - my_z)

        barrier = pltpu.get_barrier_semaphore()
        for nbr in (ypartner, xneighbor, zneighbor):
            pl.semaphore_signal(
                barrier, inc=1, device_id=nbr,
                device_id_type=pl.DeviceIdType.MESH,
            )
        pl.semaphore_wait(barrier, 3)

        my_start = my_y * S_HALF
        other_start = (1 - my_y) * S_HALF
        q_mine = (2 * my_x + my_z) * QR
        q_x = (2 * (1 - my_x) + my_z) * QR
        q_z = (2 * my_x + (1 - my_z)) * QR
        q_d = (2 * (1 - my_x) + (1 - my_z)) * QR

        def _dot(b, start, rows):
            return lax.dot_general(
                o_ref[b, :, pl.ds(start, rows)], wo_ref[:, :],
                (((0,), (0,)), ((), ())),
                preferred_element_type=jnp.float32,
            )

        y_rdmas = []
        for b in range(B):
            part = _dot(b, other_start + my_x * 2 * QR, 2 * QR)
            ysend[b] = part.astype(jnp.bfloat16)
            r = pltpu.make_async_remote_copy(
                src_ref=ysend.at[b, pl.ds(my_z * QR, QR), :],
                dst_ref=yrecv.at[b],
                send_sem=ys_sems.at[b],
                recv_sem=yr_sems.at[b],
                device_id=ypartner,
                device_id_type=pl.DeviceIdType.MESH,
            )
            r.start()
            y_rdmas.append(r)

        for b in range(B):
            out_ref[b] = _dot(b, my_start, S_HALF)

        x_rdmas, z_rdmas = [], []
        for b in range(B):
            y_rdmas[b].wait_recv()
            xr = pltpu.make_async_remote_copy(
                src_ref=yrecv.at[b], dst_ref=xrecv.at[b],
                send_sem=xs_sems.at[b], recv_sem=xr_sems.at[b],
                device_id=xneighbor, device_id_type=pl.DeviceIdType.MESH,
            )
            xr.start()
            x_rdmas.append(xr)
            zr = pltpu.make_async_remote_copy(
                src_ref=yrecv.at[b], dst_ref=zrecv.at[b],
                send_sem=zs_sems.at[b], recv_sem=zr_sems.at[b],
                device_id=zneighbor, device_id_type=pl.DeviceIdType.MESH,
            )
            zr.start()
            z_rdmas.append(zr)
            out_ref[b, pl.ds(q_mine, QR), :] = (
                out_ref[b, pl.ds(q_mine, QR), :] + yrecv[b].astype(jnp.float32)
            )

        d_rdmas = []
        for b in range(B):
            x_rdmas[b].wait_recv()
            z_rdmas[b].wait_recv()
            if b % 2 == 0:
                dr = pltpu.make_async_remote_copy(
                    src_ref=zrecv.at[b], dst_ref=drecv.at[b],
                    send_sem=ds_sems.at[b], recv_sem=dr_sems.at[b],
                    device_id=xneighbor, device_id_type=pl.DeviceIdType.MESH,
                )
            else:
                dr = pltpu.make_async_remote_copy(
                    src_ref=xrecv.at[b], dst_ref=drecv.at[b],
                    send_sem=ds_sems.at[b], recv_sem=dr_sems.at[b],
                    device_id=zneighbor, device_id_type=pl.DeviceIdType.MESH,
                )
            dr.start()
            d_rdmas.append(dr)
            out_ref[b, pl.ds(q_x, QR), :] = (
                out_ref[b, pl.ds(q_x, QR), :] + xrecv[b].astype(jnp.float32)
            )
            out_ref[b, pl.ds(q_z, QR), :] = (
                out_ref[b, pl.ds(q_z, QR), :] + zrecv[b].astype(jnp.float32)
            )

        for b in range(B):
            d_rdmas[b].wait_recv()
            out_ref[b, pl.ds(q_d, QR), :] = (
                out_ref[b, pl.ds(q_d, QR), :] + drecv[b].astype(jnp.float32)
            )

        for b in range(B):
            y_rdmas[b].wait_send()
            x_rdmas[b].wait_send()
            z_rdmas[b].wait_send()
            d_rdmas[b].wait_send()

    return pl.pallas_call(
        body,
        out_shape=jax.ShapeDtypeStruct((B, S_HALF, N), jnp.float32),
        in_specs=[
            pl.BlockSpec(memory_space=pltpu.VMEM),
            pl.BlockSpec(memory_space=pltpu.VMEM),
        ],
        out_specs=pl.BlockSpec(memory_space=pltpu.VMEM),
        scratch_shapes=[
            pltpu.VMEM((B, 2 * QR, N), jnp.bfloat16),
            pltpu.VMEM((B, QR, N), jnp.bfloat16),
            pltpu.VMEM((B, QR, N), jnp.bfloat16),
            pltpu.VMEM((B, QR, N), jnp.bfloat16),
            pltpu.VMEM((B, QR, N), jnp.bfloat16),
            pltpu.SemaphoreType.DMA((B,)),
            pltpu.SemaphoreType.DMA((B,)),
            pltpu.SemaphoreType.DMA((B,)),
            pltpu.SemaphoreType.DMA((B,)),
            pltpu.SemaphoreType.DMA((B,)),
            pltpu.SemaphoreType.DMA((B,)),
            pltpu.SemaphoreType.DMA((B,)),
            pltpu.SemaphoreType.DMA((B,)),
        ],
        compiler_params=pltpu.CompilerParams(collective_id=0),
    )(O_t, Wo)


# device time: 23721 ns/iter; 1.1345x vs baseline; 1.0086x over previous
import jax
import jax.numpy as jnp
from jax import lax
from jax.experimental import pallas as pl
from jax.experimental.pallas import tpu as pltpu

B, S, N = 4, 512, 1024
H, D = 8, 64
K = H * D
S_HALF = S // 2
QR = 64


def kernel(O, Wo):
    O_t = jnp.transpose(O, (0, 2, 3, 1)).reshape(B, K, S)

    def body(o_ref, wo_ref, out_ref, ysend, yrecv, xrecv, zrecv, drecv,
             ys_sems, yr_sems, xs_sems, xr_sems, zs_sems, zr_sems,
             ds_sems, dr_sems):
        my_x = lax.axis_index("x")
        my_y = lax.axis_index("y")
        my_z = lax.axis_index("z")
        ypartner = (my_x, 1 - my_y, my_z)
        xneighbor = (1 - my_x, my_y, my_z)
        zneighbor = (my_x, my_y, 1 - my_z)

        barrier = pltpu.get_barrier_semaphore()
        for nbr in (ypartner, xneighbor, zneighbor):
            pl.semaphore_signal(
                barrier, inc=1, device_id=nbr,
                device_id_type=pl.DeviceIdType.MESH,
            )
        pl.semaphore_wait(barrier, 3)

        my_start = my_y * S_HALF
        other_start = (1 - my_y) * S_HALF
        q_mine = (2 * my_x + my_z) * QR
        q_x = (2 * (1 - my_x) + my_z) * QR
        q_z = (2 * my_x + (1 - my_z)) * QR
        q_d = (2 * (1 - my_x) + (1 - my_z)) * QR

        def _dot(b, start, rows):
            return lax.dot_general(
                o_ref[b, :, pl.ds(start, rows)], wo_ref[:, :],
                (((0,), (0,)), ((), ())),
                preferred_element_type=jnp.float32,
            )

        y_rdmas = []
        for b in range(B):
            part = _dot(b, other_start + my_x * 2 * QR, 2 * QR)
            ysend[b] = part.astype(jnp.bfloat16)
            r = pltpu.make_async_remote_copy(
                src_ref=ysend.at[b, pl.ds(my_z * QR, QR), :],
                dst_ref=yrecv.at[b],
                send_sem=ys_sems.at[b],
                recv_sem=yr_sems.at[b],
                device_id=ypartner,
                device_id_type=pl.DeviceIdType.MESH,
            )
            r.start()
            y_rdmas.append(r)

        x_rdmas, z_rdmas = [], []
        for b in range(B):
            out_ref[b] = _dot(b, my_start, S_HALF)
            y_rdmas[b].wait_recv()
            xr = pltpu.make_async_remote_copy(
                src_ref=yrecv.at[b], dst_ref=xrecv.at[b],
                send_sem=xs_sems.at[b], recv_sem=xr_sems.at[b],
                device_id=xneighbor, device_id_type=pl.DeviceIdType.MESH,
            )
            xr.start()
            x_rdmas.append(xr)
            zr = pltpu.make_async_remote_copy(
                src_ref=yrecv.at[b], dst_ref=zrecv.at[b],
                send_sem=zs_sems.at[b], recv_sem=zr_sems.at[b],
                device_id=zneighbor, device_id_type=pl.DeviceIdType.MESH,
            )
            zr.start()
            z_rdmas.append(zr)
            out_ref[b, pl.ds(q_mine, QR), :] = (
                out_ref[b, pl.ds(q_mine, QR), :] + yrecv[b].astype(jnp.float32)
            )

        d_rdmas = []
        for b in range(B):
            x_rdmas[b].wait_recv()
            z_rdmas[b].wait_recv()
            if b % 2 == 0:
                dr = pltpu.make_async_remote_copy(
                    src_ref=zrecv.at[b], dst_ref=drecv.at[b],
                    send_sem=ds_sems.at[b], recv_sem=dr_sems.at[b],
                    device_id=xneighbor, device_id_type=pl.DeviceIdType.MESH,
                )
            else:
                dr = pltpu.make_async_remote_copy(
                    src_ref=xrecv.at[b], dst_ref=drecv.at[b],
                    send_sem=ds_sems.at[b], recv_sem=dr_sems.at[b],
                    device_id=zneighbor, device_id_type=pl.DeviceIdType.MESH,
                )
            dr.start()
            d_rdmas.append(dr)
            out_ref[b, pl.ds(q_x, QR), :] = (
                out_ref[b, pl.ds(q_x, QR), :] + xrecv[b].astype(jnp.float32)
            )
            out_ref[b, pl.ds(q_z, QR), :] = (
                out_ref[b, pl.ds(q_z, QR), :] + zrecv[b].astype(jnp.float32)
            )

        for b in range(B):
            d_rdmas[b].wait_recv()
            out_ref[b, pl.ds(q_d, QR), :] = (
                out_ref[b, pl.ds(q_d, QR), :] + drecv[b].astype(jnp.float32)
            )

        for b in range(B):
            y_rdmas[b].wait_send()
            x_rdmas[b].wait_send()
            z_rdmas[b].wait_send()
            d_rdmas[b].wait_send()

    return pl.pallas_call(
        body,
        out_shape=jax.ShapeDtypeStruct((B, S_HALF, N), jnp.float32),
        in_specs=[
            pl.BlockSpec(memory_space=pltpu.VMEM),
            pl.BlockSpec(memory_space=pltpu.VMEM),
        ],
        out_specs=pl.BlockSpec(memory_space=pltpu.VMEM),
        scratch_shapes=[
            pltpu.VMEM((B, 2 * QR, N), jnp.bfloat16),
            pltpu.VMEM((B, QR, N), jnp.bfloat16),
            pltpu.VMEM((B, QR, N), jnp.bfloat16),
            pltpu.VMEM((B, QR, N), jnp.bfloat16),
            pltpu.VMEM((B, QR, N), jnp.bfloat16),
            pltpu.SemaphoreType.DMA((B,)),
            pltpu.SemaphoreType.DMA((B,)),
            pltpu.SemaphoreType.DMA((B,)),
            pltpu.SemaphoreType.DMA((B,)),
            pltpu.SemaphoreType.DMA((B,)),
            pltpu.SemaphoreType.DMA((B,)),
            pltpu.SemaphoreType.DMA((B,)),
            pltpu.SemaphoreType.DMA((B,)),
        ],
        compiler_params=pltpu.CompilerParams(collective_id=0),
    )(O_t, Wo)


# device time: 21874 ns/iter; 1.2303x vs baseline; 1.0844x over previous
import jax
import jax.numpy as jnp
from jax import lax
from jax.experimental import pallas as pl
from jax.experimental.pallas import tpu as pltpu

B, S, N = 4, 512, 1024
H, D = 8, 64
K = H * D
S_HALF = S // 2
QR = 64


def kernel(O, Wo):
    O_t = jnp.transpose(O, (0, 2, 3, 1)).reshape(B, K, S)
    O_t = pltpu.with_memory_space_constraint(O_t, pltpu.MemorySpace.HBM)
    Wo = pltpu.with_memory_space_constraint(Wo, pltpu.MemorySpace.HBM)

    def body(o_hbm, wo_hbm, out_ref, wo_ref, o1, o2,
             ysend, yrecv, xrecv, zrecv, drecv,
             wo_sem, o1_sems, o2_sems,
             ys_sems, yr_sems, xs_sems, xr_sems, zs_sems, zr_sems,
             ds_sems, dr_sems):
        my_x = lax.axis_index("x")
        my_y = lax.axis_index("y")
        my_z = lax.axis_index("z")
        ypartner = (my_x, 1 - my_y, my_z)
        xneighbor = (1 - my_x, my_y, my_z)
        zneighbor = (my_x, my_y, 1 - my_z)

        my_start0 = my_y * S_HALF
        other_start0 = (1 - my_y) * S_HALF

        wo_copy = pltpu.make_async_copy(wo_hbm, wo_ref, wo_sem)
        wo_copy.start()
        o1_copies, o2_copies = [], []
        for b in range(B):
            c = pltpu.make_async_copy(
                o_hbm.at[b, :, pl.ds(other_start0 + my_x * 2 * QR, 2 * QR)],
                o1.at[b], o1_sems.at[b],
            )
            c.start()
            o1_copies.append(c)
        for b in range(B):
            c = pltpu.make_async_copy(
                o_hbm.at[b, :, pl.ds(my_start0, S_HALF)],
                o2.at[b], o2_sems.at[b],
            )
            c.start()
            o2_copies.append(c)

        barrier = pltpu.get_barrier_semaphore()
        for nbr in (ypartner, xneighbor, zneighbor):
            pl.semaphore_signal(
                barrier, inc=1, device_id=nbr,
                device_id_type=pl.DeviceIdType.MESH,
            )
        pl.semaphore_wait(barrier, 3)

        wo_copy.wait()

        q_mine = (2 * my_x + my_z) * QR
        q_x = (2 * (1 - my_x) + my_z) * QR
        q_z = (2 * my_x + (1 - my_z)) * QR
        q_d = (2 * (1 - my_x) + (1 - my_z)) * QR

        def _dot(src_ref, b):
            return lax.dot_general(
                src_ref[b], wo_ref[:, :],
                (((0,), (0,)), ((), ())),
                preferred_element_type=jnp.float32,
            )

        y_rdmas = []
        for b in range(B):
            o1_copies[b].wait()
            part = _dot(o1, b)
            ysend[b] = part.astype(jnp.bfloat16)
            r = pltpu.make_async_remote_copy(
                src_ref=ysend.at[b, pl.ds(my_z * QR, QR), :],
                dst_ref=yrecv.at[b],
                send_sem=ys_sems.at[b],
                recv_sem=yr_sems.at[b],
                device_id=ypartner,
                device_id_type=pl.DeviceIdType.MESH,
            )
            r.start()
            y_rdmas.append(r)

        x_rdmas, z_rdmas = [], []
        for b in range(B):
            o2_copies[b].wait()
            out_ref[b] = _dot(o2, b)
            y_rdmas[b].wait_recv()
            xr = pltpu.make_async_remote_copy(
                src_ref=yrecv.at[b], dst_ref=xrecv.at[b],
                send_sem=xs_sems.at[b], recv_sem=xr_sems.at[b],
                device_id=xneighbor, device_id_type=pl.DeviceIdType.MESH,
            )
            xr.start()
            x_rdmas.append(xr)
            zr = pltpu.make_async_remote_copy(
                src_ref=yrecv.at[b], dst_ref=zrecv.at[b],
                send_sem=zs_sems.at[b], recv_sem=zr_sems.at[b],
                device_id=zneighbor, device_id_type=pl.DeviceIdType.MESH,
            )
            zr.start()
            z_rdmas.append(zr)
            out_ref[b, pl.ds(q_mine, QR), :] = (
                out_ref[b, pl.ds(q_mine, QR), :] + yrecv[b].astype(jnp.float32)
            )

        d_rdmas = []
        for b in range(B):
            x_rdmas[b].wait_recv()
            z_rdmas[b].wait_recv()
            if b % 2 == 0:
                dr = pltpu.make_async_remote_copy(
                    src_ref=zrecv.at[b], dst_ref=drecv.at[b],
                    send_sem=ds_sems.at[b], recv_sem=dr_sems.at[b],
                    device_id=xneighbor, device_id_type=pl.DeviceIdType.MESH,
                )
            else:
                dr = pltpu.make_async_remote_copy(
                    src_ref=xrecv.at[b], dst_ref=drecv.at[b],
                    send_sem=ds_sems.at[b], recv_sem=dr_sems.at[b],
                    device_id=zneighbor, device_id_type=pl.DeviceIdType.MESH,
                )
            dr.start()
            d_rdmas.append(dr)
            out_ref[b, pl.ds(q_x, QR), :] = (
                out_ref[b, pl.ds(q_x, QR), :] + xrecv[b].astype(jnp.float32)
            )
            out_ref[b, pl.ds(q_z, QR), :] = (
                out_ref[b, pl.ds(q_z, QR), :] + zrecv[b].astype(jnp.float32)
            )

        for b in range(B):
            d_rdmas[b].wait_recv()
            out_ref[b, pl.ds(q_d, QR), :] = (
                out_ref[b, pl.ds(q_d, QR), :] + drecv[b].astype(jnp.float32)
            )

        for b in range(B):
            y_rdmas[b].wait_send()
            x_rdmas[b].wait_send()
            z_rdmas[b].wait_send()
            d_rdmas[b].wait_send()

    return pl.pallas_call(
        body,
        out_shape=jax.ShapeDtypeStruct((B, S_HALF, N), jnp.float32),
        in_specs=[
            pl.BlockSpec(memory_space=pltpu.MemorySpace.HBM),
            pl.BlockSpec(memory_space=pltpu.MemorySpace.HBM),
        ],
        out_specs=pl.BlockSpec(memory_space=pltpu.VMEM),
        scratch_shapes=[
            pltpu.VMEM((K, N), jnp.float32),
            pltpu.VMEM((B, K, 2 * QR), jnp.float32),
            pltpu.VMEM((B, K, S_HALF), jnp.float32),
            pltpu.VMEM((B, 2 * QR, N), jnp.bfloat16),
            pltpu.VMEM((B, QR, N), jnp.bfloat16),
            pltpu.VMEM((B, QR, N), jnp.bfloat16),
            pltpu.VMEM((B, QR, N), jnp.bfloat16),
            pltpu.VMEM((B, QR, N), jnp.bfloat16),
            pltpu.SemaphoreType.DMA,
            pltpu.SemaphoreType.DMA((B,)),
            pltpu.SemaphoreType.DMA((B,)),
            pltpu.SemaphoreType.DMA((B,)),
            pltpu.SemaphoreType.DMA((B,)),
            pltpu.SemaphoreType.DMA((B,)),
            pltpu.SemaphoreType.DMA((B,)),
            pltpu.SemaphoreType.DMA((B,)),
            pltpu.SemaphoreType.DMA((B,)),
            pltpu.SemaphoreType.DMA((B,)),
            pltpu.SemaphoreType.DMA((B,)),
        ],
        compiler_params=pltpu.CompilerParams(collective_id=0),
    )(O_t, Wo)
